# baseline (speedup 1.0000x reference)
import functools

import jax
import jax.numpy as jnp
from jax import lax
from jax.experimental import pallas as pl
from jax.experimental.pallas import tpu as pltpu
from jax.experimental.pallas import tpu_sc as plsc

N = 10000
E = 320000
IN = 128
MID = 128
NREG = 16

NTILES = 32
CHUNK = 128
DEG_NCHUNK = 79
SCH = 112
NCHUNK = 93
NBUF = 3
ET_PAD = NTILES * NCHUNK * SCH
ED_PAD = NTILES * DEG_NCHUNK * CHUNK
N2 = 10240
ROWS_PER_TILE = N2 // 16

@functools.lru_cache(maxsize=None)
def _build_deg_kernel():
    mesh = plsc.VectorSubcoreMesh(core_axis_name="c", subcore_axis_name="s")
    nch = DEG_NCHUNK
    zwords = N2 // 16

    @functools.partial(
        pl.kernel,
        mesh=mesh,
        out_type=jax.ShapeDtypeStruct((2, N2), jnp.float32),
        scratch_types=[
            pltpu.VMEM((nch, CHUNK), jnp.int32),
            pltpu.VMEM((128,), jnp.float32),
            pltpu.VMEM((zwords,), jnp.float32),
            pltpu.VMEM_SHARED((N2,), jnp.float32),
        ],
    )
    def deg_kernel(dst_hbm, out_hbm, dst_v, ones_v, zer_v, hist_sh):
        c = lax.axis_index("c")
        s = lax.axis_index("s")
        wid = s * 2 + c
        pltpu.sync_copy(dst_hbm.at[wid], dst_v)

        def fill(i, carry):
            ones_v[pl.ds(i * 16, 16)] = jnp.ones((16,), jnp.float32)
            zer_v[pl.ds(i * 16, 16)] = jnp.zeros((16,), jnp.float32)
            return carry

        lax.fori_loop(0, 8, fill, 0)

        def zmore(i, carry):
            zer_v[pl.ds(i * 16, 16)] = jnp.zeros((16,), jnp.float32)
            return carry

        lax.fori_loop(8, zwords // 16, zmore, 0)

        pltpu.sync_copy(zer_v, hist_sh.at[pl.ds(s * zwords, zwords)])
        plsc.subcore_barrier()

        def accum(j, carry):
            pltpu.sync_copy(ones_v, hist_sh.at[dst_v.at[j]], add=True)
            return carry

        lax.fori_loop(0, nch, accum, 0)
        plsc.subcore_barrier()
        pltpu.sync_copy(hist_sh.at[pl.ds(s * zwords, zwords)],
                        out_hbm.at[c, pl.ds(s * zwords, zwords)])

    return deg_kernel


@functools.lru_cache(maxsize=None)
def _build_scatter_kernel():
    mesh = plsc.VectorSubcoreMesh(core_axis_name="c", subcore_axis_name="s")

    @functools.partial(
        pl.kernel,
        mesh=mesh,
        out_type=jax.ShapeDtypeStruct((2, N2, IN), jnp.float32),
        scratch_types=[
            pltpu.VMEM((NBUF, 2, SCH), jnp.int32),
            pltpu.VMEM((NBUF, SCH, IN), jnp.float32),
            pltpu.VMEM_SHARED((N2, IN), jnp.float32),
            pltpu.SemaphoreType.DMA,
            pltpu.SemaphoreType.DMA,
            pltpu.SemaphoreType.DMA,
            pltpu.SemaphoreType.DMA,
            pltpu.SemaphoreType.DMA,
            pltpu.SemaphoreType.DMA,
        ],
    )
    def scatter_kernel(idx_hbm, g_hbm, out_hbm,
                       idx_v, bufs_v, acc_sh, sg0, sg1, sg2, ss0, ss1, ss2):
        c = lax.axis_index("c")
        s = lax.axis_index("s")
        wid = s * 2 + c
        sgs = (sg0, sg1, sg2)
        sss = (ss0, ss1, ss2)

        def zrow(i, carry):
            n = i // (IN // 16)
            b = n // SCH
            r = n % SCH
            k = i % (IN // 16)
            bufs_v[b, r, pl.ds(k * 16, 16)] = jnp.zeros((16,), jnp.float32)
            return carry

        lax.fori_loop(0, NBUF * SCH * (IN // 16), zrow, 0)

        base = s * ROWS_PER_TILE
        for off in (0, 112, 224, 336, 448, 528):
            pltpu.sync_copy(bufs_v.at[0], acc_sh.at[pl.ds(base + off, SCH)])

        def pipeline(g_ref):
            for b in range(NBUF):
                pltpu.sync_copy(idx_hbm.at[wid, b], idx_v.at[b])
            pltpu.async_copy(g_ref.at[idx_v.at[0, 0]], bufs_v.at[0], sg0)
            pltpu.async_copy(g_ref.at[idx_v.at[1, 0]], bufs_v.at[1], sg1)
            pltpu.async_copy(bufs_v.at[2], acc_sh.at[idx_v.at[2, 1]], ss2,
                             add=True)
            plsc.subcore_barrier()

            def group(i, carry):
                for b in range(NBUF):
                    k = i * NBUF + b
                    b1 = (b + 2) % NBUF
                    pltpu.make_async_copy(
                        g_ref.at[pl.ds(0, SCH)], bufs_v.at[b], sgs[b]).wait()
                    pltpu.async_copy(
                        bufs_v.at[b], acc_sh.at[idx_v.at[b, 1]], sss[b],
                        add=True)
                    pltpu.make_async_copy(
                        g_ref.at[pl.ds(0, SCH)], bufs_v.at[b1], sss[b1]).wait()
                    kp = jnp.minimum(k + 2, NCHUNK - 1)
                    pltpu.sync_copy(idx_hbm.at[wid, kp], idx_v.at[b1])
                    pltpu.async_copy(g_ref.at[idx_v.at[b1, 0]], bufs_v.at[b1],
                                     sgs[b1])
                return carry

            lax.fori_loop(0, NCHUNK // NBUF, group, 0)
            pltpu.make_async_copy(
                g_ref.at[pl.ds(0, SCH)], bufs_v.at[0], sg0).wait()
            pltpu.make_async_copy(
                g_ref.at[pl.ds(0, SCH)], bufs_v.at[1], sg1).wait()
            pltpu.make_async_copy(
                g_ref.at[pl.ds(0, SCH)], bufs_v.at[2], ss2).wait()
            plsc.subcore_barrier()

        @pl.when(c == 0)
        def _():
            pipeline(g_hbm.at[0])

        @pl.when(c == 1)
        def _():
            pipeline(g_hbm.at[1])
        pltpu.sync_copy(acc_sh.at[pl.ds(base, ROWS_PER_TILE)],
                        out_hbm.at[c, pl.ds(base, ROWS_PER_TILE)])

    return scatter_kernel


def _tc1(x, conv_w, hist):
    B = 1024

    def body(x_ref, w_ref, hist_ref, g_ref):
        deg = 1.0 + hist_ref[0] + hist_ref[1]
        dinv = lax.rsqrt(deg)
        h = jnp.dot(x_ref[...], w_ref[...], preferred_element_type=jnp.float32)
        v = h * dinv[:, None]
        g_ref[0] = v
        g_ref[1] = v

    return pl.pallas_call(
        body,
        grid=(N2 // B,),
        in_specs=[
            pl.BlockSpec((B, IN), lambda i: (i, 0)),
            pl.BlockSpec((IN, IN), lambda i: (0, 0)),
            pl.BlockSpec((2, B), lambda i: (0, i)),
        ],
        out_specs=pl.BlockSpec((2, B, IN), lambda i: (0, i, 0)),
        out_shape=jax.ShapeDtypeStruct((2, N2, IN), jnp.float32),
    )(x, conv_w, hist)


def _tc2(acc2, hist, x, conv_b, w1, b1, w2, b2, w3, b3, w4, b4, bwf, bb):
    B = 1024

    def body(acc_ref, hist_ref, x_ref, cb, w1r, b1r, w2r, b2r, w3r, b3r,
             w4r, b4r, bwr, bbr, y1_ref, y2_ref):
        deg = 1.0 + hist_ref[0] + hist_ref[1]
        dinv = lax.rsqrt(deg)
        accs = acc_ref[0] + acc_ref[1]
        conv = accs * dinv[:, None] + cb[...]
        out = jnp.maximum(conv, 0.0)
        x0 = out + x_ref[...]
        x0 = jnp.maximum(
            jnp.dot(x0, w1r[...], preferred_element_type=jnp.float32) + b1r[...], 0.0)
        x1 = jnp.maximum(
            jnp.dot(x0, w2r[...], preferred_element_type=jnp.float32) + b2r[...], 0.0)
        y1_ref[...] = jnp.dot(x1, w3r[...], preferred_element_type=jnp.float32) + b3r[...]
        x2 = jnp.maximum(
            jnp.dot(x0, w4r[...], preferred_element_type=jnp.float32) + b4r[...], 0.0)
        t = jnp.dot(x2, bwr[...], preferred_element_type=jnp.float32)
        t3 = t.reshape(B, NREG, MID)
        y2_ref[...] = jnp.sum(t3 * x2[:, None, :], axis=2) + bbr[...]

    return pl.pallas_call(
        body,
        grid=(N2 // B,),
        in_specs=[
            pl.BlockSpec((2, B, IN), lambda i: (0, i, 0)),
            pl.BlockSpec((2, B), lambda i: (0, i)),
            pl.BlockSpec((B, IN), lambda i: (i, 0)),
            pl.BlockSpec((1, IN), lambda i: (0, 0)),
            pl.BlockSpec((IN, MID), lambda i: (0, 0)),
            pl.BlockSpec((1, MID), lambda i: (0, 0)),
            pl.BlockSpec((MID, MID), lambda i: (0, 0)),
            pl.BlockSpec((1, MID), lambda i: (0, 0)),
            pl.BlockSpec((MID, 1), lambda i: (0, 0)),
            pl.BlockSpec((1, 1), lambda i: (0, 0)),
            pl.BlockSpec((MID, MID), lambda i: (0, 0)),
            pl.BlockSpec((1, MID), lambda i: (0, 0)),
            pl.BlockSpec((MID, NREG * MID), lambda i: (0, 0)),
            pl.BlockSpec((1, NREG), lambda i: (0, 0)),
        ],
        out_specs=[
            pl.BlockSpec((B, 1), lambda i: (i, 0)),
            pl.BlockSpec((B, NREG), lambda i: (i, 0)),
        ],
        out_shape=[
            jax.ShapeDtypeStruct((N2, 1), jnp.float32),
            jax.ShapeDtypeStruct((N2, NREG), jnp.float32),
        ],
    )(acc2, hist, x, conv_b, w1, b1, w2, b2, w3, b3, w4, b4, bwf, bb)


def kernel(x, edge_index, conv_w, conv_b, w1, b1, w2, b2, w3, b3, w4, b4, bw, bb):
    src = edge_index[0]
    dst = edge_index[1]

    x_pad = jnp.concatenate([x, jnp.zeros((N2 - N, IN), jnp.float32)], axis=0)
    dstd = jnp.concatenate([dst, jnp.full((ED_PAD - E,), N, jnp.int32)])
    hist = _build_deg_kernel()(dstd.reshape(NTILES, DEG_NCHUNK, CHUNK))
    g_pad = _tc1(x_pad, conv_w, hist)

    loop = jnp.arange(N, dtype=jnp.int32)
    pad_n = ET_PAD - E - N
    srcp = jnp.concatenate(
        [src, loop, jnp.full((pad_n,), N, jnp.int32)]).reshape(NTILES, NCHUNK, SCH)
    dstp = jnp.concatenate(
        [dst, loop, jnp.zeros((pad_n,), jnp.int32)]).reshape(NTILES, NCHUNK, SCH)
    idxp = jnp.stack([srcp, dstp], axis=2)

    acc2 = _build_scatter_kernel()(idxp, g_pad)

    bwf = jnp.transpose(bw, (1, 0, 2)).reshape(MID, NREG * MID)
    y1, y2 = _tc2(acc2, hist, x_pad, conv_b.reshape(1, IN), w1, b1.reshape(1, MID),
                  w2, b2.reshape(1, MID), w3, b3.reshape(1, 1), w4,
                  b4.reshape(1, MID), bwf, bb.reshape(1, NREG))
    return (y1[:N], y2[:N])


# speedup vs baseline: 1.1191x; 1.1191x over previous
import functools

import jax
import jax.numpy as jnp
from jax import lax
from jax.experimental import pallas as pl
from jax.experimental.pallas import tpu as pltpu
from jax.experimental.pallas import tpu_sc as plsc

N = 10000
E = 320000
IN = 128
MID = 128
NREG = 16

NTILES = 32
CHUNK = 128
DEG_NCHUNK = 79
SCH = 112
NCHUNK = 93
NCH0 = 120
NCH1 = 66
NBUF = 3
ET_PAD = NTILES * NCHUNK * SCH
ED_PAD = NTILES * DEG_NCHUNK * CHUNK
N2 = 10240
ROWS_PER_TILE = N2 // 16

@functools.lru_cache(maxsize=None)
def _build_deg_kernel():
    mesh = plsc.VectorSubcoreMesh(core_axis_name="c", subcore_axis_name="s")
    nch = DEG_NCHUNK
    zwords = N2 // 16

    @functools.partial(
        pl.kernel,
        mesh=mesh,
        out_type=jax.ShapeDtypeStruct((2, N2), jnp.float32),
        scratch_types=[
            pltpu.VMEM((nch, CHUNK), jnp.int32),
            pltpu.VMEM((128,), jnp.float32),
            pltpu.VMEM((zwords,), jnp.float32),
            pltpu.VMEM_SHARED((N2,), jnp.float32),
        ],
    )
    def deg_kernel(dst_hbm, out_hbm, dst_v, ones_v, zer_v, hist_sh):
        c = lax.axis_index("c")
        s = lax.axis_index("s")
        wid = s * 2 + c
        pltpu.sync_copy(dst_hbm.at[wid], dst_v)

        def fill(i, carry):
            ones_v[pl.ds(i * 16, 16)] = jnp.ones((16,), jnp.float32)
            zer_v[pl.ds(i * 16, 16)] = jnp.zeros((16,), jnp.float32)
            return carry

        lax.fori_loop(0, 8, fill, 0)

        def zmore(i, carry):
            zer_v[pl.ds(i * 16, 16)] = jnp.zeros((16,), jnp.float32)
            return carry

        lax.fori_loop(8, zwords // 16, zmore, 0)

        pltpu.sync_copy(zer_v, hist_sh.at[pl.ds(s * zwords, zwords)])
        plsc.subcore_barrier()

        def accum(j, carry):
            pltpu.sync_copy(ones_v, hist_sh.at[dst_v.at[j]], add=True)
            return carry

        lax.fori_loop(0, nch, accum, 0)
        plsc.subcore_barrier()
        pltpu.sync_copy(hist_sh.at[pl.ds(s * zwords, zwords)],
                        out_hbm.at[c, pl.ds(s * zwords, zwords)])

    return deg_kernel


@functools.lru_cache(maxsize=None)
def _build_scatter_kernel():
    mesh = plsc.VectorSubcoreMesh(core_axis_name="c", subcore_axis_name="s")

    @functools.partial(
        pl.kernel,
        mesh=mesh,
        out_type=jax.ShapeDtypeStruct((2, N2, IN), jnp.float32),
        scratch_types=[
            pltpu.VMEM((NBUF, 2, SCH), jnp.int32),
            pltpu.VMEM((NBUF, SCH, IN), jnp.float32),
            pltpu.VMEM_SHARED((N2, IN), jnp.float32),
            pltpu.SemaphoreType.DMA,
            pltpu.SemaphoreType.DMA,
            pltpu.SemaphoreType.DMA,
            pltpu.SemaphoreType.DMA,
            pltpu.SemaphoreType.DMA,
            pltpu.SemaphoreType.DMA,
        ],
    )
    def scatter_kernel(idx_hbm, g_hbm, out_hbm,
                       idx_v, bufs_v, acc_sh, sg0, sg1, sg2, ss0, ss1, ss2):
        c = lax.axis_index("c")
        s = lax.axis_index("s")
        wid = s * 2 + c
        sgs = (sg0, sg1, sg2)
        sss = (ss0, ss1, ss2)

        def zrow(i, carry):
            n = i // (IN // 16)
            b = n // SCH
            r = n % SCH
            k = i % (IN // 16)
            bufs_v[b, r, pl.ds(k * 16, 16)] = jnp.zeros((16,), jnp.float32)
            return carry

        lax.fori_loop(0, NBUF * SCH * (IN // 16), zrow, 0)

        base = s * ROWS_PER_TILE
        for off in (0, 112, 224, 336, 448, 528):
            pltpu.sync_copy(bufs_v.at[0], acc_sh.at[pl.ds(base + off, SCH)])

        def pipeline(start, nch):
            for b in range(NBUF):
                pltpu.sync_copy(idx_hbm.at[start + b], idx_v.at[b])
            pltpu.async_copy(g_hbm.at[idx_v.at[0, 0]], bufs_v.at[0], sg0)
            pltpu.async_copy(g_hbm.at[idx_v.at[1, 0]], bufs_v.at[1], sg1)
            pltpu.async_copy(bufs_v.at[2], acc_sh.at[idx_v.at[2, 1]], ss2,
                             add=True)
            plsc.subcore_barrier()

            def group(i, carry):
                for b in range(NBUF):
                    k = i * NBUF + b
                    b1 = (b + 2) % NBUF
                    pltpu.make_async_copy(
                        g_hbm.at[pl.ds(0, SCH)], bufs_v.at[b], sgs[b]).wait()
                    pltpu.async_copy(
                        bufs_v.at[b], acc_sh.at[idx_v.at[b, 1]], sss[b],
                        add=True)
                    pltpu.make_async_copy(
                        g_hbm.at[pl.ds(0, SCH)], bufs_v.at[b1], sss[b1]).wait()
                    kp = start + jnp.minimum(k + 2, nch - 1)
                    pltpu.sync_copy(idx_hbm.at[kp], idx_v.at[b1])
                    pltpu.async_copy(g_hbm.at[idx_v.at[b1, 0]], bufs_v.at[b1],
                                     sgs[b1])
                return carry

            lax.fori_loop(0, nch // NBUF, group, 0)
            pltpu.make_async_copy(
                g_hbm.at[pl.ds(0, SCH)], bufs_v.at[0], sg0).wait()
            pltpu.make_async_copy(
                g_hbm.at[pl.ds(0, SCH)], bufs_v.at[1], sg1).wait()
            pltpu.make_async_copy(
                g_hbm.at[pl.ds(0, SCH)], bufs_v.at[2], ss2).wait()
            plsc.subcore_barrier()

        @pl.when(c == 0)
        def _():
            pipeline(s * (NCH0 + NCH1), NCH0)

        @pl.when(c == 1)
        def _():
            pipeline(s * (NCH0 + NCH1) + NCH0, NCH1)
        pltpu.sync_copy(acc_sh.at[pl.ds(base, ROWS_PER_TILE)],
                        out_hbm.at[c, pl.ds(base, ROWS_PER_TILE)])

    return scatter_kernel


def _tc1(x, conv_w, hist):
    B = 1024

    def body(x_ref, w_ref, hist_ref, g_ref):
        deg = 1.0 + hist_ref[0] + hist_ref[1]
        dinv = lax.rsqrt(deg)
        h = jnp.dot(x_ref[...], w_ref[...], preferred_element_type=jnp.float32)
        g_ref[...] = h * dinv[:, None]

    return pl.pallas_call(
        body,
        grid=(N2 // B,),
        in_specs=[
            pl.BlockSpec((B, IN), lambda i: (i, 0)),
            pl.BlockSpec((IN, IN), lambda i: (0, 0)),
            pl.BlockSpec((2, B), lambda i: (0, i)),
        ],
        out_specs=pl.BlockSpec((B, IN), lambda i: (i, 0)),
        out_shape=jax.ShapeDtypeStruct((N2, IN), jnp.float32),
    )(x, conv_w, hist)


def _tc2(acc2, hist, x, conv_b, w1, b1, w2, b2, w3, b3, w4, b4, bwf, bb):
    B = 1024

    def body(acc_ref, hist_ref, x_ref, cb, w1r, b1r, w2r, b2r, w3r, b3r,
             w4r, b4r, bwr, bbr, y1_ref, y2_ref):
        deg = 1.0 + hist_ref[0] + hist_ref[1]
        dinv = lax.rsqrt(deg)
        accs = acc_ref[0] + acc_ref[1]
        conv = accs * dinv[:, None] + cb[...]
        out = jnp.maximum(conv, 0.0)
        x0 = out + x_ref[...]
        x0 = jnp.maximum(
            jnp.dot(x0, w1r[...], preferred_element_type=jnp.float32) + b1r[...], 0.0)
        x1 = jnp.maximum(
            jnp.dot(x0, w2r[...], preferred_element_type=jnp.float32) + b2r[...], 0.0)
        y1_ref[...] = jnp.dot(x1, w3r[...], preferred_element_type=jnp.float32) + b3r[...]
        x2 = jnp.maximum(
            jnp.dot(x0, w4r[...], preferred_element_type=jnp.float32) + b4r[...], 0.0)
        t = jnp.dot(x2, bwr[...], preferred_element_type=jnp.float32)
        t3 = t.reshape(B, NREG, MID)
        y2_ref[...] = jnp.sum(t3 * x2[:, None, :], axis=2) + bbr[...]

    return pl.pallas_call(
        body,
        grid=(N2 // B,),
        in_specs=[
            pl.BlockSpec((2, B, IN), lambda i: (0, i, 0)),
            pl.BlockSpec((2, B), lambda i: (0, i)),
            pl.BlockSpec((B, IN), lambda i: (i, 0)),
            pl.BlockSpec((1, IN), lambda i: (0, 0)),
            pl.BlockSpec((IN, MID), lambda i: (0, 0)),
            pl.BlockSpec((1, MID), lambda i: (0, 0)),
            pl.BlockSpec((MID, MID), lambda i: (0, 0)),
            pl.BlockSpec((1, MID), lambda i: (0, 0)),
            pl.BlockSpec((MID, 1), lambda i: (0, 0)),
            pl.BlockSpec((1, 1), lambda i: (0, 0)),
            pl.BlockSpec((MID, MID), lambda i: (0, 0)),
            pl.BlockSpec((1, MID), lambda i: (0, 0)),
            pl.BlockSpec((MID, NREG * MID), lambda i: (0, 0)),
            pl.BlockSpec((1, NREG), lambda i: (0, 0)),
        ],
        out_specs=[
            pl.BlockSpec((B, 1), lambda i: (i, 0)),
            pl.BlockSpec((B, NREG), lambda i: (i, 0)),
        ],
        out_shape=[
            jax.ShapeDtypeStruct((N2, 1), jnp.float32),
            jax.ShapeDtypeStruct((N2, NREG), jnp.float32),
        ],
    )(acc2, hist, x, conv_b, w1, b1, w2, b2, w3, b3, w4, b4, bwf, bb)


def kernel(x, edge_index, conv_w, conv_b, w1, b1, w2, b2, w3, b3, w4, b4, bw, bb):
    src = edge_index[0]
    dst = edge_index[1]

    x_pad = jnp.concatenate([x, jnp.zeros((N2 - N, IN), jnp.float32)], axis=0)
    dstd = jnp.concatenate([dst, jnp.full((ED_PAD - E,), N, jnp.int32)])
    hist = _build_deg_kernel()(dstd.reshape(NTILES, DEG_NCHUNK, CHUNK))
    g_pad = _tc1(x_pad, conv_w, hist)

    loop = jnp.arange(N, dtype=jnp.int32)
    pad_n = ET_PAD - E - N
    srcp = jnp.concatenate(
        [src, loop, jnp.full((pad_n,), N, jnp.int32)]).reshape(NTILES * NCHUNK, SCH)
    dstp = jnp.concatenate(
        [dst, loop, jnp.zeros((pad_n,), jnp.int32)]).reshape(NTILES * NCHUNK, SCH)
    idxp = jnp.stack([srcp, dstp], axis=1)

    acc2 = _build_scatter_kernel()(idxp, g_pad)

    bwf = jnp.transpose(bw, (1, 0, 2)).reshape(MID, NREG * MID)
    y1, y2 = _tc2(acc2, hist, x_pad, conv_b.reshape(1, IN), w1, b1.reshape(1, MID),
                  w2, b2.reshape(1, MID), w3, b3.reshape(1, 1), w4,
                  b4.reshape(1, MID), bwf, bb.reshape(1, NREG))
    return (y1[:N], y2[:N])
